# Initial kernel scaffold; baseline (speedup 1.0000x reference)
import jax
import jax.numpy as jnp
from jax.experimental import pallas as pl


def kernel(x, gate_w, gate_b, w_gate_up, b_gate_up, w_down, b_down):
    raise NotImplementedError("write your pallas kernel here")


# speedup vs baseline: 1.1873x; 1.1873x over previous
import functools

import jax
import jax.numpy as jnp
from jax.experimental import pallas as pl
from jax.experimental.pallas import tpu as pltpu

N_EXPERTS = 8
D_MODEL = 1024
D_FF = 2048
ALPHA = 1.702
BETA = 1.0
LIMIT = 7.0

BM = 512
FB = 1024
NF = D_FF // FB


def _routing_weights(x, gw, gb, bm):
    logits = jax.lax.dot_general(
        x, gw, (((1,), (1,)), ((), ())),
        preferred_element_type=jnp.float32) + gb
    lane = jax.lax.broadcasted_iota(jnp.int32, (bm, N_EXPERTS), 1)
    m1 = jnp.max(logits, axis=1, keepdims=True)
    i1 = jnp.min(jnp.where(logits == m1, lane, N_EXPERTS), axis=1,
                 keepdims=True)
    masked = jnp.where(lane == i1, -jnp.inf, logits)
    m2 = jnp.max(masked, axis=1, keepdims=True)
    i2 = jnp.min(jnp.where(masked == m2, lane, N_EXPERTS), axis=1,
                 keepdims=True)
    w1 = jax.nn.sigmoid(m1 - m2)
    w2 = 1.0 - w1
    comb = (jnp.where(lane == i1, w1, 0.0)
            + jnp.where(lane == i2, w2, 0.0))
    return comb


def _moe_body(x_ref, gw_ref, gb_ref, wglu_ref, wlin_ref, bglu_ref,
              blin_ref, wd_ref, bd_ref, out_ref):
    e = pl.program_id(1)
    f = pl.program_id(2)

    x = x_ref[...]
    comb = _routing_weights(x, gw_ref[...], gb_ref[...], BM)
    lane = jax.lax.broadcasted_iota(jnp.int32, (BM, N_EXPERTS), 1)
    ce = jnp.sum(jnp.where(lane == e, comb, 0.0), axis=1,
                 keepdims=True)

    h_glu = jax.lax.dot_general(
        x, wglu_ref[...], (((1,), (0,)), ((), ())),
        preferred_element_type=jnp.float32) + bglu_ref[...]
    h_lin = jax.lax.dot_general(
        x, wlin_ref[...], (((1,), (0,)), ((), ())),
        preferred_element_type=jnp.float32) + blin_ref[...]
    h_glu = jnp.minimum(h_glu, LIMIT)
    h_lin = jnp.clip(h_lin, -LIMIT, LIMIT)
    act = h_glu * jax.nn.sigmoid(ALPHA * h_glu) * (h_lin + BETA)

    y = jax.lax.dot_general(
        act, wd_ref[...], (((1,), (0,)), ((), ())),
        preferred_element_type=jnp.float32)

    @pl.when(jnp.logical_and(e == 0, f == 0))
    def _():
        out_ref[...] = jnp.zeros_like(out_ref)

    contrib = ce * y

    @pl.when(f == 0)
    def _():
        out_ref[...] += ce * bd_ref[...]

    out_ref[...] += contrib


@jax.jit
def kernel(x, gate_w, gate_b, w_gate_up, b_gate_up, w_down, b_down):
    t_tokens = x.shape[0]
    gb2 = gate_b.reshape(1, N_EXPERTS)
    bgu3 = b_gate_up.reshape(N_EXPERTS, 1, 2 * D_FF)
    bd3 = b_down.reshape(N_EXPERTS, 1, D_MODEL)

    grid = (t_tokens // BM, N_EXPERTS, NF)
    out = pl.pallas_call(
        _moe_body,
        grid=grid,
        in_specs=[
            pl.BlockSpec((BM, D_MODEL), lambda t, e, f: (t, 0)),
            pl.BlockSpec((N_EXPERTS, D_MODEL), lambda t, e, f: (0, 0)),
            pl.BlockSpec((1, N_EXPERTS), lambda t, e, f: (0, 0)),
            pl.BlockSpec((None, D_MODEL, FB), lambda t, e, f: (e, 0, f)),
            pl.BlockSpec((None, D_MODEL, FB),
                         lambda t, e, f: (e, 0, f + NF)),
            pl.BlockSpec((None, 1, FB), lambda t, e, f: (e, 0, f)),
            pl.BlockSpec((None, 1, FB), lambda t, e, f: (e, 0, f + NF)),
            pl.BlockSpec((None, FB, D_MODEL), lambda t, e, f: (e, f, 0)),
            pl.BlockSpec((None, 1, D_MODEL), lambda t, e, f: (e, 0, 0)),
        ],
        out_specs=pl.BlockSpec((BM, D_MODEL), lambda t, e, f: (t, 0)),
        out_shape=jax.ShapeDtypeStruct((t_tokens, D_MODEL), jnp.float32),
    )(x, gate_w, gb2, w_gate_up, w_gate_up, bgu3, bgu3, w_down, bd3)
    return out
